# baseline (speedup 1.0000x reference)
import functools

import jax
import jax.numpy as jnp
from jax import lax
from jax.experimental import pallas as pl
from jax.experimental.pallas import tpu as pltpu
from jax.experimental.pallas import tpu_sc as plsc

NUM_ROWS = 4096 * 26
NUM_COLS = 1000
NC = 2
NS = 16
NW = NC * NS
ROWS_PER_W = NUM_ROWS // NW
LANES = 16
ROWS_PER_BUF = 32
GROUPS = ROWS_PER_BUF // LANES
NBUF = 3
NCHUNKS = ROWS_PER_W // ROWS_PER_BUF
FULL_ROUNDS = NCHUNKS // NBUF
TAIL = NCHUNKS - FULL_ROUNDS * NBUF
BUF_WORDS = ROWS_PER_BUF * NUM_COLS
assert ROWS_PER_W % ROWS_PER_BUF == 0 and ROWS_PER_BUF % LANES == 0

_mesh = plsc.VectorSubcoreMesh(core_axis_name="c", subcore_axis_name="s")


@functools.partial(
    pl.kernel,
    out_type=jax.ShapeDtypeStruct((NUM_ROWS * NUM_COLS,), jnp.float32),
    mesh=_mesh,
    scratch_types=(
        [pltpu.VMEM((ROWS_PER_W,), jnp.int32)]
        + [pltpu.VMEM((BUF_WORDS,), jnp.float32) for _ in range(NBUF)]
        + [pltpu.SemaphoreType.DMA for _ in range(NBUF)]
    ),
    compiler_params=pltpu.CompilerParams(needs_layout_passes=False),
)
def _one_hot_sc(x_hbm, out_hbm, idx_v, *rest):
    bufs = list(rest[:NBUF])
    sems = list(rest[NBUF:])
    wid = lax.axis_index("s") * NC + lax.axis_index("c")
    base_row = wid * ROWS_PER_W

    pltpu.sync_copy(x_hbm.at[pl.ds(base_row, ROWS_PER_W)], idx_v)

    zeros16 = jnp.zeros((LANES,), jnp.float32)
    ones16 = jnp.ones((LANES,), jnp.float32)
    row_off = lax.iota(jnp.int32, 16) * NUM_COLS

    def _zero(i, carry):
        for b in range(NBUF):
            bufs[b][pl.ds(i * LANES, LANES)] = zeros16
        return carry

    lax.fori_loop(0, BUF_WORDS // LANES, _zero, 0)

    def scatter(b, c, val16):
        for g in range(GROUPS):
            idx = idx_v[pl.ds(c * ROWS_PER_BUF + g * LANES, LANES)]
            plsc.store_scatter(
                bufs[b], [g * LANES * NUM_COLS + row_off + idx], val16)

    def dma(b, c):
        dst = out_hbm.at[
            pl.ds((base_row + c * ROWS_PER_BUF) * NUM_COLS, BUF_WORDS)]
        return pltpu.make_async_copy(bufs[b], dst, sems[b])

    for b in range(NBUF):
        scatter(b, b, ones16)
        dma(b, b).start()

    def step(g, carry):
        for b in range(NBUF):
            c = g * NBUF + b
            dma(b, c).wait()
            scatter(b, c - NBUF, zeros16)
            scatter(b, c, ones16)
            dma(b, c).start()
        return carry

    lax.fori_loop(1, FULL_ROUNDS, step, 0)

    for j in range(TAIL):
        c = FULL_ROUNDS * NBUF + j
        dma(j, c).wait()
        scatter(j, c - NBUF, zeros16)
        scatter(j, c, ones16)
        dma(j, c).start()

    for b in range(NBUF):
        dma(b, 0).wait()


def kernel(x):
    x = x.reshape(-1).astype(jnp.int32)
    out = _one_hot_sc(x)
    return out.reshape(4096, 26, NUM_COLS)
